# baseline (speedup 1.0000x reference)
import functools
import math

import jax
import jax.numpy as jnp
from jax import lax
from jax.experimental import pallas as pl
from jax.experimental.pallas import tpu as pltpu
from jax.experimental.pallas import tpu_sc as plsc

D_MODEL = 128
LANES = 16
CHUNK = 128


def _emb_kernel(n_rows: int):
    info = plsc.get_sparse_core_info()
    num_workers = info.num_cores * info.num_subcores
    per_worker = n_rows // num_workers
    n_chunks = per_worker // CHUNK
    scale = jnp.float32(math.sqrt(D_MODEL))

    mesh = plsc.VectorSubcoreMesh(core_axis_name="c", subcore_axis_name="s")

    @functools.partial(
        pl.kernel,
        mesh=mesh,
        out_type=jax.ShapeDtypeStruct((n_rows, D_MODEL), jnp.float32),
        scratch_types=[
            pltpu.VMEM((CHUNK,), jnp.int32),
            pltpu.VMEM((CHUNK, D_MODEL), jnp.float32),
            pltpu.SemaphoreType.DMA,
        ],
    )
    def k(idx_hbm, table_hbm, out_hbm, idx_v, rows_v, sem):
        wid = lax.axis_index("s") * info.num_cores + lax.axis_index("c")
        base = wid * per_worker

        def chunk_body(c, carry):
            off = base + c * CHUNK
            pltpu.sync_copy(idx_hbm.at[pl.ds(off, CHUNK)], idx_v)
            pltpu.async_copy(table_hbm.at[idx_v], rows_v, sem).wait()

            def scale_row(r, carry2):
                for j in range(D_MODEL // LANES):
                    sl = pl.ds(j * LANES, LANES)
                    rows_v[r, sl] = rows_v[r, sl] * scale
                return carry2

            lax.fori_loop(0, CHUNK, scale_row, 0, unroll=False)
            pltpu.sync_copy(rows_v, out_hbm.at[pl.ds(off, CHUNK)])
            return carry

        lax.fori_loop(0, n_chunks, chunk_body, 0, unroll=False)

    return k


def kernel(input_batch, table):
    b, t = input_batch.shape
    n_rows = b * t
    idx = input_batch.reshape(n_rows)
    out = _emb_kernel(n_rows)(idx, table)
    return out.reshape(b, t, D_MODEL)


# speedup vs baseline: 2.0159x; 2.0159x over previous
import functools
import math

import jax
import jax.numpy as jnp
from jax import lax
from jax.experimental import pallas as pl
from jax.experimental.pallas import tpu as pltpu
from jax.experimental.pallas import tpu_sc as plsc

D_MODEL = 128
LANES = 16
CHUNK = 256
GSUB = 128


def _emb_kernel(n_rows: int):
    info = plsc.get_sparse_core_info()
    num_workers = info.num_cores * info.num_subcores
    per_worker = n_rows // num_workers
    n_chunks = per_worker // CHUNK
    n_iter = n_chunks // 2
    scale = jnp.float32(math.sqrt(D_MODEL))

    mesh = plsc.VectorSubcoreMesh(core_axis_name="c", subcore_axis_name="s")

    @functools.partial(
        pl.kernel,
        mesh=mesh,
        out_type=jax.ShapeDtypeStruct((n_rows, D_MODEL), jnp.float32),
        scratch_types=[
            pltpu.VMEM((per_worker,), jnp.int32),
            pltpu.VMEM((CHUNK, D_MODEL), jnp.float32),
            pltpu.VMEM((CHUNK, D_MODEL), jnp.float32),
            pltpu.SemaphoreType.DMA,
            pltpu.SemaphoreType.DMA,
            pltpu.SemaphoreType.DMA,
            pltpu.SemaphoreType.DMA,
        ],
    )
    def k(idx_hbm, table_hbm, out_hbm, idx_all, rows0, rows1,
          gsem0, gsem1, osem0, osem1):
        wid = lax.axis_index("s") * info.num_cores + lax.axis_index("c")
        base = wid * per_worker
        rows = (rows0, rows1)
        gsem = (gsem0, gsem1)
        osem = (osem0, osem1)

        pltpu.sync_copy(idx_hbm.at[pl.ds(base, per_worker)], idx_all)

        def start_gather(c, b):
            off = c * CHUNK
            for j in range(CHUNK // GSUB):
                pltpu.async_copy(
                    table_hbm.at[idx_all.at[pl.ds(off + j * GSUB, GSUB)]],
                    rows[b].at[pl.ds(j * GSUB, GSUB)],
                    gsem[b])

        def wait_gather(b):
            for j in range(CHUNK // GSUB):
                pltpu.make_async_copy(
                    table_hbm.at[idx_all.at[pl.ds(j * GSUB, GSUB)]],
                    rows[b].at[pl.ds(j * GSUB, GSUB)],
                    gsem[b]).wait()

        def start_out(c, b):
            pltpu.async_copy(rows[b], out_hbm.at[pl.ds(base + c * CHUNK, CHUNK)],
                             osem[b])

        def wait_out(b):
            pltpu.make_async_copy(rows[b], out_hbm.at[pl.ds(base, CHUNK)],
                                  osem[b]).wait()

        def scale_rows(b):
            def srow(r, carry):
                for j in range(D_MODEL // LANES):
                    sl = pl.ds(j * LANES, LANES)
                    rows[b][r, sl] = rows[b][r, sl] * scale
                return carry
            lax.fori_loop(0, CHUNK, srow, 0, unroll=2)

        start_gather(0, 0)

        def body(i, carry):
            c0 = 2 * i

            @pl.when(i > 0)
            def _():
                wait_out(1)
            start_gather(c0 + 1, 1)
            wait_gather(0)
            scale_rows(0)
            start_out(c0, 0)

            wait_out(0)

            @pl.when(i < n_iter - 1)
            def _():
                start_gather(c0 + 2, 0)
            wait_gather(1)
            scale_rows(1)
            start_out(c0 + 1, 1)
            return carry

        lax.fori_loop(0, n_iter, body, 0, unroll=False)
        wait_out(1)

    return k


def kernel(input_batch, table):
    b, t = input_batch.shape
    n_rows = b * t
    idx = input_batch.reshape(n_rows)
    out = _emb_kernel(n_rows)(idx, table)
    return out.reshape(b, t, D_MODEL)


# speedup vs baseline: 2.0435x; 1.0137x over previous
import functools
import math

import jax
import jax.numpy as jnp
from jax import lax
from jax.experimental import pallas as pl
from jax.experimental.pallas import tpu as pltpu
from jax.experimental.pallas import tpu_sc as plsc

D_MODEL = 128
LANES = 16
CHUNK = 128
NBUF = 4


def _emb_kernel(n_rows: int):
    info = plsc.get_sparse_core_info()
    num_workers = info.num_cores * info.num_subcores
    per_worker = n_rows // num_workers
    n_chunks = per_worker // CHUNK
    n_iter = n_chunks // NBUF
    scale = jnp.float32(math.sqrt(D_MODEL))

    mesh = plsc.VectorSubcoreMesh(core_axis_name="c", subcore_axis_name="s")

    @functools.partial(
        pl.kernel,
        mesh=mesh,
        out_type=jax.ShapeDtypeStruct((n_rows, D_MODEL), jnp.float32),
        scratch_types=[
            pltpu.VMEM((per_worker,), jnp.int32),
        ] + [pltpu.VMEM((CHUNK, D_MODEL), jnp.float32)] * NBUF
          + [pltpu.SemaphoreType.DMA] * (2 * NBUF),
    )
    def k(idx_hbm, table_hbm, out_hbm, idx_all, *bufs_and_sems):
        rows = bufs_and_sems[:NBUF]
        gsem = bufs_and_sems[NBUF:2 * NBUF]
        osem = bufs_and_sems[2 * NBUF:]
        wid = lax.axis_index("s") * info.num_cores + lax.axis_index("c")
        base = wid * per_worker

        pltpu.sync_copy(idx_hbm.at[pl.ds(base, per_worker)], idx_all)

        def start_gather(c, b):
            pltpu.async_copy(
                table_hbm.at[idx_all.at[pl.ds(c * CHUNK, CHUNK)]],
                rows[b], gsem[b])

        def wait_gather(b):
            pltpu.make_async_copy(
                table_hbm.at[idx_all.at[pl.ds(0, CHUNK)]],
                rows[b], gsem[b]).wait()

        def start_out(c, b):
            pltpu.async_copy(rows[b], out_hbm.at[pl.ds(base + c * CHUNK, CHUNK)],
                             osem[b])

        def wait_out(b):
            pltpu.make_async_copy(rows[b], out_hbm.at[pl.ds(base, CHUNK)],
                                  osem[b]).wait()

        def scale_rows(b):
            def srow(r, carry):
                for j in range(D_MODEL // LANES):
                    sl = pl.ds(j * LANES, LANES)
                    rows[b][r, sl] = rows[b][r, sl] * scale
                return carry
            lax.fori_loop(0, CHUNK, srow, 0, unroll=2)

        start_gather(0, 0)
        start_gather(1, 1)

        def body(i, carry):
            for b in range(NBUF):
                c = NBUF * i + b
                bn = (b + 2) % NBUF

                @pl.when(c >= 2)
                def _():
                    wait_out(bn)

                @pl.when(c + 2 < n_chunks)
                def _():
                    start_gather(c + 2, bn)
                wait_gather(b)
                scale_rows(b)
                start_out(c, b)
            return carry

        lax.fori_loop(0, n_iter, body, 0, unroll=False)
        wait_out((n_chunks - 2) % NBUF)
        wait_out((n_chunks - 1) % NBUF)

    return k


def kernel(input_batch, table):
    b, t = input_batch.shape
    n_rows = b * t
    idx = input_batch.reshape(n_rows)
    out = _emb_kernel(n_rows)(idx, table)
    return out.reshape(b, t, D_MODEL)
